# baseline (speedup 1.0000x reference)
import functools

import jax
import jax.numpy as jnp
from jax import lax
from jax.experimental import pallas as pl
from jax.experimental.pallas import tpu as pltpu
from jax.experimental.pallas import tpu_sc as plsc

VOCAB_SIZE = 1000000
EMBED_DIM = 64
CONTEXT_LEN = 200
BATCH = 1024

_NUM_CORES = 2
_NUM_SUBCORES = 16
_NUM_WORKERS = _NUM_CORES * _NUM_SUBCORES
_BPW = BATCH // _NUM_WORKERS

_NTC = (VOCAB_SIZE + 127) // 128
_VPAD = _NTC * 128

_mesh = plsc.VectorSubcoreMesh(core_axis_name="c", subcore_axis_name="s")


@functools.partial(
    pl.kernel,
    mesh=_mesh,
    compiler_params=pltpu.CompilerParams(
        use_tc_tiling_on_sc=True, needs_layout_passes=False,
        disable_bounds_checks=True),
    out_type=jax.ShapeDtypeStruct((VOCAB_SIZE * (EMBED_DIM // 2),), jnp.int32),
    scratch_types=[
        pltpu.VMEM((EMBED_DIM, 512), jnp.float32),
        pltpu.VMEM((EMBED_DIM, 512), jnp.float32),
        pltpu.VMEM((512 * (EMBED_DIM // 2),), jnp.int32),
        pltpu.VMEM((512 * (EMBED_DIM // 2),), jnp.int32),
        pltpu.SemaphoreType.DMA,
        pltpu.SemaphoreType.DMA,
        pltpu.SemaphoreType.DMA,
        pltpu.SemaphoreType.DMA,
    ],
)
def _transpose_kernel(tokt_hbm, out_hbm, stage0, stage1, trbuf0, trbuf1,
                      is0, is1, ws0, ws1):
    wid = lax.axis_index("s") * _NUM_CORES + lax.axis_index("c")
    ng = (_NTC - 1) // 4
    per = (ng + _NUM_WORKERS - 1) // _NUM_WORKERS
    lo = wid * per
    hi = jnp.minimum(lo + per, ng)

    lane = lax.iota(jnp.int32, 16)
    lbase = [(c * 16 + lane) * (EMBED_DIM // 2) for c in range(8)]

    GW = 4 * 128
    GWORDS = GW * (EMBED_DIM // 2)

    def start_in(g, stage, sem):
        start = pl.multiple_of(g * GW, 128)
        pltpu.async_copy(tokt_hbm.at[:, pl.ds(start, GW)], stage, sem)

    def wait_in(g, stage, sem):
        start = pl.multiple_of(g * GW, 128)
        pltpu.make_async_copy(tokt_hbm.at[:, pl.ds(start, GW)], stage,
                              sem).wait()

    def transpose_tc(stage, trbuf, col0, obase):
        KB = 8
        for c in range(8):
            tcol = col0 + c * 16
            abase = obase + lbase[c]
            for k0 in range(0, EMBED_DIM // 2, KB):
                xs = [(stage[2 * k, pl.ds(tcol, 16)],
                       stage[2 * k + 1, pl.ds(tcol, 16)])
                      for k in range(k0, k0 + KB)]
                ws = [plsc.bitcast(
                          plsc.pack(x0, x1, format=plsc.PackFormat.INTERLEAVED),
                          jnp.int32) for (x0, x1) in xs]
                for i, w in enumerate(ws):
                    plsc.store_scatter(trbuf, [abase + (k0 + i)], w)

    def transpose(stage, trbuf):
        def tbody(t, carry):
            transpose_tc(stage, trbuf, t * 128, t * 4096)
            return carry
        lax.fori_loop(0, 4, tbody, 0)

    def start_out(g, trbuf, sem):
        pltpu.async_copy(trbuf, out_hbm.at[pl.ds(g * GWORDS, GWORDS)], sem)

    def wait_out(g, trbuf, sem):
        pltpu.make_async_copy(trbuf, out_hbm.at[pl.ds(g * GWORDS, GWORDS)],
                              sem).wait()

    start_in(lo, stage0, is0)

    def body(i, carry):
        g0 = lo + 2 * i
        g1 = g0 + 1

        @pl.when(g1 < hi)
        def _():
            start_in(g1, stage1, is1)

        @pl.when(g0 < hi)
        def _():
            wait_in(g0, stage0, is0)

            @pl.when(i >= 1)
            def _():
                wait_out(g0 - 2, trbuf0, ws0)

            transpose(stage0, trbuf0)
            start_out(g0, trbuf0, ws0)

            @pl.when(g0 + 2 < hi)
            def _():
                start_in(g0 + 2, stage0, is0)

        @pl.when(g1 < hi)
        def _():
            wait_in(g1, stage1, is1)

            @pl.when(i >= 1)
            def _():
                wait_out(g1 - 2, trbuf1, ws1)

            transpose(stage1, trbuf1)
            start_out(g1, trbuf1, ws1)

        return carry

    nit = (per + 1) // 2
    lax.fori_loop(0, nit, body, 0)

    @pl.when(hi - 2 >= lo)
    def _():
        wait_out(hi - 2, trbuf0, ws0)

    @pl.when(hi - 1 >= lo)
    def _():
        wait_out(hi - 1, trbuf1, ws1)

    @pl.when(wid == _NUM_WORKERS - 1)
    def _():
        def tail_tc(tc, full):
            start = pl.multiple_of(tc * 128, 128)
            pltpu.sync_copy(tokt_hbm.at[:, pl.ds(start, 128)],
                            stage0.at[:, pl.ds(0, 128)])
            transpose_tc(stage0, trbuf0, 0, 0)
            n = 4096 if full else 2048
            pltpu.sync_copy(trbuf0.at[pl.ds(0, n)],
                            out_hbm.at[pl.ds(tc * 4096, n)])

        for j in range(4 * ((_NTC - 1) // 4), _NTC - 1):
            tail_tc(j, True)
        tail_tc(_NTC - 1, False)


@functools.partial(
    pl.kernel,
    mesh=_mesh,
    compiler_params=pltpu.CompilerParams(
        use_tc_tiling_on_sc=False, needs_layout_passes=False),
    out_type=jax.ShapeDtypeStruct((CONTEXT_LEN * BATCH, EMBED_DIM), jnp.float32),
    scratch_types=[
        pltpu.VMEM((CONTEXT_LEN, _BPW), jnp.int32),
        pltpu.VMEM((CONTEXT_LEN, EMBED_DIM), jnp.float32),
        pltpu.VMEM((_BPW, EMBED_DIM // 2), jnp.int32),
        pltpu.VMEM((_BPW, EMBED_DIM // 2), jnp.int32),
        pltpu.VMEM((_BPW, EMBED_DIM // 2), jnp.int32),
        pltpu.VMEM((_BPW, EMBED_DIM // 2), jnp.int32),
        pltpu.VMEM((_BPW, EMBED_DIM), jnp.float32),
        pltpu.VMEM((_BPW, EMBED_DIM), jnp.float32),
        pltpu.VMEM((_BPW, EMBED_DIM), jnp.float32),
        pltpu.VMEM((_BPW, EMBED_DIM), jnp.float32),
        pltpu.SemaphoreType.DMA,
        pltpu.SemaphoreType.DMA,
        pltpu.SemaphoreType.DMA,
        pltpu.SemaphoreType.DMA,
        pltpu.SemaphoreType.DMA,
        pltpu.SemaphoreType.DMA,
        pltpu.SemaphoreType.DMA,
        pltpu.SemaphoreType.DMA,
    ],
)
def _gather_kernel(idx_hbm, tok_hbm, pos_hbm, out_hbm,
                   idx_v, pos_v, raw0, raw1, raw2, raw3,
                   rows0, rows1, rows2, rows3,
                   gs0, gs1, gs2, gs3, os0, os1, os2, os3):
    wid = lax.axis_index("s") * _NUM_CORES + lax.axis_index("c")
    b0 = wid * _BPW

    pltpu.sync_copy(idx_hbm.at[:, pl.ds(b0, _BPW)], idx_v)
    pltpu.sync_copy(pos_hbm, pos_v)

    lane = lax.iota(jnp.int32, 16)
    ev = 2 * lane
    od = 2 * lane + 1

    def add_pos(l, raw, rows):
        lv = jnp.full((16,), l, jnp.int32)
        pvs = [plsc.load_gather(pos_v, [lv, h * 32 + eo])
               for h in range(2) for eo in (ev, od)]
        for j in range(_BPW):
            for h in range(2):
                w = raw[j, pl.ds(h * 16, 16)]
                bf = plsc.bitcast(w, jnp.bfloat16)
                a, b = plsc.unpack(bf, format=plsc.PackFormat.INTERLEAVED)
                a = a + pvs[2 * h]
                b = b + pvs[2 * h + 1]
                jv = jnp.full((16,), j, jnp.int32)
                plsc.store_scatter(rows, [jv, h * 32 + ev], a)
                plsc.store_scatter(rows, [jv, h * 32 + od], b)

    def out_slice(l):
        return out_hbm.at[pl.ds(l * BATCH + b0, _BPW)]

    raws = [raw0, raw1, raw2, raw3]
    rowss = [rows0, rows1, rows2, rows3]
    gss = [gs0, gs1, gs2, gs3]
    oss = [os0, os1, os2, os3]

    for p in range(4):
        pltpu.async_copy(tok_hbm.at[idx_v.at[p]], raws[p], gss[p])

    def body(l4, carry):
        lb = 4 * l4
        for p in range(4):
            l = lb + p
            pltpu.make_async_copy(tok_hbm.at[idx_v.at[l]], raws[p],
                                  gss[p]).wait()

            @pl.when(l4 >= 1)
            def _():
                pltpu.make_async_copy(rowss[p], out_slice(l), oss[p]).wait()

            add_pos(l, raws[p], rowss[p])
            pltpu.async_copy(rowss[p], out_slice(l), oss[p])

            @pl.when(l4 < CONTEXT_LEN // 4 - 1)
            def _():
                pltpu.async_copy(tok_hbm.at[idx_v.at[l + 4]], raws[p], gss[p])

        return carry

    lax.fori_loop(0, CONTEXT_LEN // 4, body, 0)
    for p in range(4):
        pltpu.make_async_copy(rowss[p], out_slice(CONTEXT_LEN - 4 + p),
                              oss[p]).wait()


def kernel(inputs, token_table, position_table):
    tok_t = jnp.transpose(token_table)
    flat = _transpose_kernel(tok_t)
    tok_lin = flat.reshape(VOCAB_SIZE, EMBED_DIM // 2)
    idx_t = jnp.transpose(inputs).astype(jnp.int32)
    out = _gather_kernel(idx_t, tok_lin, position_table)
    return jnp.transpose(out.reshape(CONTEXT_LEN, BATCH, EMBED_DIM), (1, 0, 2))
